# Initial kernel scaffold; baseline (speedup 1.0000x reference)
import jax
import jax.numpy as jnp
from jax.experimental import pallas as pl


def kernel(x, adj_mat_indices, adj_mat_values, weights, bias):
    raise NotImplementedError("write your pallas kernel here")


# speedup vs baseline: 4.5706x; 4.5706x over previous
import functools

import jax
import jax.numpy as jnp
from jax import lax
from jax.experimental import pallas as pl
from jax.experimental.pallas import tpu as pltpu
from jax.experimental.pallas import tpu_sc as plsc

N_NODES_ = 10000
N_EDGES_ = 320000
D_ = 128

NUM_CORES = 2
NUM_SUBCORES = 16
NUM_WORKERS = NUM_CORES * NUM_SUBCORES
EDGES_PER_WORKER = N_EDGES_ // NUM_WORKERS
CHUNK = 80
NUM_CHUNKS = EDGES_PER_WORKER // CHUNK
RBLK = 80
NUM_RBLKS = N_NODES_ // RBLK
LANES = 16
DSUB = D_ // LANES


def _sc_spmm(x, row, col, val):
    mesh = plsc.VectorSubcoreMesh(
        core_axis_name="c", subcore_axis_name="s")

    @functools.partial(
        pl.kernel,
        out_type=jax.ShapeDtypeStruct((NUM_CORES, N_NODES_, D_), jnp.float32),
        mesh=mesh,
        scratch_types=dict(
            acc=pltpu.VMEM_SHARED((N_NODES_, D_), jnp.float32),
            row_v=pltpu.VMEM((CHUNK,), jnp.int32),
            col_v=pltpu.VMEM((CHUNK,), jnp.int32),
            val_v=pltpu.VMEM((CHUNK,), jnp.float32),
            rows_v=pltpu.VMEM((CHUNK, D_), jnp.float32),
            zbuf=pltpu.VMEM((RBLK, D_), jnp.float32),
            sem=pltpu.SemaphoreType.DMA,
        ),
    )
    def spmm(x_hbm, row_hbm, col_hbm, val_hbm, out_hbm,
             acc, row_v, col_v, val_v, rows_v, zbuf, sem):
        cid = lax.axis_index("c")
        sid = lax.axis_index("s")
        wid = sid * NUM_CORES + cid

        zeros16 = jnp.zeros((LANES,), jnp.float32)

        def zfill(i, _):
            for j in range(DSUB):
                zbuf[i, pl.ds(j * LANES, LANES)] = zeros16
            return _

        lax.fori_loop(0, RBLK, zfill, None)

        def zcopy(i, _):
            b = sid + i * NUM_SUBCORES

            @pl.when(b < NUM_RBLKS)
            def _do():
                pltpu.sync_copy(zbuf, acc.at[pl.ds(b * RBLK, RBLK)])

            return _

        lax.fori_loop(0, pl.cdiv(NUM_RBLKS, NUM_SUBCORES), zcopy, None)
        plsc.subcore_barrier()

        base_e = wid * EDGES_PER_WORKER

        def chunk_body(c, _):
            e0 = base_e + c * CHUNK
            pltpu.sync_copy(row_hbm.at[pl.ds(e0, CHUNK)], row_v)
            pltpu.sync_copy(col_hbm.at[pl.ds(e0, CHUNK)], col_v)
            pltpu.sync_copy(val_hbm.at[pl.ds(e0, CHUNK)], val_v)
            pltpu.async_copy(x_hbm.at[col_v], rows_v, sem).wait()

            def scale(g, _):
                vv = val_v[pl.ds(g * LANES, LANES)]
                for k in range(LANES):
                    e = g * LANES + k
                    s = vv[k]
                    for j in range(DSUB):
                        sl = pl.ds(j * LANES, LANES)
                        rows_v[e, sl] = rows_v[e, sl] * s
                return _

            lax.fori_loop(0, CHUNK // LANES, scale, None)
            pltpu.sync_copy(rows_v, acc.at[row_v], add=True)
            return _

        lax.fori_loop(0, NUM_CHUNKS, chunk_body, None)
        plsc.subcore_barrier()

        def ocopy(i, _):
            b = sid + i * NUM_SUBCORES

            @pl.when(b < NUM_RBLKS)
            def _do():
                pltpu.sync_copy(acc.at[pl.ds(b * RBLK, RBLK)],
                                out_hbm.at[cid, pl.ds(b * RBLK, RBLK)])

            return _

        lax.fori_loop(0, pl.cdiv(NUM_RBLKS, NUM_SUBCORES), ocopy, None)

    return spmm(x, row, col, val)


def _tc_linear(partials, weights, bias2d):
    blk = 2000

    def body(p_ref, w_ref, b_ref, o_ref):
        agg = p_ref[0] + p_ref[1]
        o_ref[...] = (
            jnp.dot(agg, w_ref[...], preferred_element_type=jnp.float32)
            + b_ref[...])

    return pl.pallas_call(
        body,
        out_shape=jax.ShapeDtypeStruct((N_NODES_, D_), jnp.float32),
        grid=(N_NODES_ // blk,),
        in_specs=[
            pl.BlockSpec((NUM_CORES, blk, D_), lambda i: (0, i, 0)),
            pl.BlockSpec((D_, D_), lambda i: (0, 0)),
            pl.BlockSpec((1, D_), lambda i: (0, 0)),
        ],
        out_specs=pl.BlockSpec((blk, D_), lambda i: (i, 0)),
    )(partials, weights, bias2d)


def kernel(x, adj_mat_indices, adj_mat_values, weights, bias):
    row = adj_mat_indices[0]
    col = adj_mat_indices[1]
    partials = _sc_spmm(x, row, col, adj_mat_values)
    return _tc_linear(partials, weights, bias.reshape(1, D_))
